# baseline (speedup 1.0000x reference)
import jax
import jax.numpy as jnp
from jax import lax
from jax.experimental import pallas as pl
from jax.experimental.pallas import tpu as pltpu
from jax.experimental.pallas import tpu_sc as plsc

PREFIX_SIZE = 1024
KV_SIZE = 16384
BATCH = 16
PREFIX_LEN = 128
B = BATCH * PREFIX_LEN
NC, NS = 2, 16
NW = NC * NS
BPW = B // NW

F = 4
DV = KV_SIZE // F
VF = PREFIX_SIZE * F
BF = B * F
SLEN = 8
ROWS_PER_CHUNK = SLEN // F
NCH = BPW // ROWS_PER_CHUNK
NBUF = 2


def _body(table_hbm, idx_hbm, out_hbm, idx_v, bufs, in_sems, out_sems):
    wid = lax.axis_index("s") * NC + lax.axis_index("c")
    fbase = wid * BPW * F

    pltpu.sync_copy(idx_hbm.at[pl.ds(fbase, BPW * F)], idx_v)

    def start_gather(cg, b):
        pltpu.async_copy(
            table_hbm.at[idx_v.at[pl.ds(cg * SLEN, SLEN)]],
            bufs.at[b],
            in_sems.at[b],
        )

    def wait_gather(b):
        pltpu.make_async_copy(
            table_hbm.at[pl.ds(0, SLEN)], bufs.at[b], in_sems.at[b]
        ).wait()

    for b in range(NBUF):
        start_gather(b, b)

    @pl.loop(0, NCH - NBUF, step=NBUF)
    def _(g0):
        for b in range(NBUF):
            cg = g0 + b
            wait_gather(b)
            pltpu.async_copy(
                bufs.at[b], out_hbm.at[pl.ds(fbase + cg * SLEN, SLEN)],
                out_sems.at[b],
            ).wait()
            start_gather(cg + NBUF, b)

    for b in range(NBUF):
        cg = NCH - NBUF + b
        wait_gather(b)
        pltpu.async_copy(
            bufs.at[b], out_hbm.at[pl.ds(fbase + cg * SLEN, SLEN)],
            out_sems.at[b],
        ).wait()


@jax.jit
def _gather(table_f, idx_flat):
    mesh = plsc.VectorSubcoreMesh(
        core_axis_name="c", subcore_axis_name="s", num_cores=NC, num_subcores=NS
    )
    f = pl.kernel(
        _body,
        out_type=jax.ShapeDtypeStruct((BF, DV), jnp.float32),
        mesh=mesh,
        scratch_types=[
            pltpu.VMEM((BPW * F,), jnp.int32),
            pltpu.VMEM((NBUF, SLEN, DV), jnp.float32),
            pltpu.SemaphoreType.DMA((NBUF,)),
            pltpu.SemaphoreType.DMA((NBUF,)),
        ],
    )
    return f(table_f, idx_flat)


def kernel(prefix, table):
    idx = prefix.reshape(B)
    idx_flat = (idx[:, None] * F + jnp.arange(F, dtype=jnp.int32)).reshape(BF)
    table_f = table.reshape(VF, DV)
    out = _gather(table_f, idx_flat)
    return out.reshape(BATCH, PREFIX_LEN, KV_SIZE)
